# baseline (speedup 1.0000x reference)
import functools

import jax
import jax.numpy as jnp
from jax import lax
from jax.experimental import pallas as pl
from jax.experimental.pallas import tpu as pltpu
from jax.experimental.pallas import tpu_sc as plsc

B = 16384
D = 32
NC = 2
NS = 16
NW = NC * NS
BPW = B // NW
CH = 128
NCH = BPW // CH
L = 16


def _sc_diff_body(user_hbm, table_hbm, pos_hbm, neg_hbm, out_hbm,
                  pidx_v, nidx_v, u_v, p_v, n_v, diff_v, sem):
    wid = lax.axis_index("s") * NC + lax.axis_index("c")
    base = wid * BPW

    pltpu.sync_copy(pos_hbm.at[wid], pidx_v)
    pltpu.sync_copy(neg_hbm.at[wid], nidx_v)
    pltpu.sync_copy(user_hbm.at[pl.ds(base, BPW)], u_v)

    copies = []
    for j in range(NCH):
        copies.append(pltpu.async_copy(
            table_hbm.at[pidx_v.at[j]], p_v.at[pl.ds(j * CH, CH)], sem))
        copies.append(pltpu.async_copy(
            table_hbm.at[nidx_v.at[j]], n_v.at[pl.ds(j * CH, CH)], sem))
    for c in copies:
        c.wait()

    lane = lax.iota(jnp.int32, L)

    def chunk(c, carry):
        rows = c * L + lane
        acc = jnp.zeros((L,), jnp.float32)
        for d in range(D):
            cols = (lane + d) & (D - 1)
            u = plsc.load_gather(u_v, [rows, cols])
            p = plsc.load_gather(p_v, [rows, cols])
            n = plsc.load_gather(n_v, [rows, cols])
            acc = acc + u * (p - n)
        diff_v[pl.ds(c * L, L)] = acc
        return carry

    lax.fori_loop(0, BPW // L, chunk, 0)

    pltpu.sync_copy(diff_v, out_hbm.at[pl.ds(base, BPW)])


_sc_diff = functools.partial(
    pl.kernel,
    mesh=plsc.VectorSubcoreMesh(core_axis_name="c", subcore_axis_name="s"),
    out_type=jax.ShapeDtypeStruct((B,), jnp.float32),
    scratch_types=[
        pltpu.VMEM((NCH, CH), jnp.int32),
        pltpu.VMEM((NCH, CH), jnp.int32),
        pltpu.VMEM((BPW, D), jnp.float32),
        pltpu.VMEM((BPW, D), jnp.float32),
        pltpu.VMEM((BPW, D), jnp.float32),
        pltpu.VMEM((BPW,), jnp.float32),
        pltpu.SemaphoreType.DMA,
    ],
    compiler_params=pltpu.CompilerParams(
        use_tc_tiling_on_sc=False,
        needs_layout_passes=False,
    ),
)(_sc_diff_body)


def _tc_loss_body(x_ref, o_ref):
    x = x_ref[...]
    y = -x
    sp = jnp.maximum(y, 0.0) + jnp.log1p(jnp.exp(-jnp.abs(y)))
    o_ref[0, 0] = jnp.sum(sp) * jnp.float32(1.0 / B)


_tc_loss = pl.pallas_call(
    _tc_loss_body,
    out_shape=jax.ShapeDtypeStruct((1, 1), jnp.float32),
    out_specs=pl.BlockSpec(memory_space=pltpu.SMEM),
)


def kernel(user_embeddings, item_embeddings, positive_item_indices,
           negative_item_indices):
    pos = positive_item_indices.astype(jnp.int32).reshape(NW, NCH, CH)
    neg = negative_item_indices.astype(jnp.int32).reshape(NW, NCH, CH)
    diff = _sc_diff(user_embeddings, item_embeddings, pos, neg)
    loss = _tc_loss(diff.reshape(B // 128, 128))
    return loss[0, 0]
